# Initial kernel scaffold; baseline (speedup 1.0000x reference)
import jax
import jax.numpy as jnp
from jax.experimental import pallas as pl


def kernel(x, edge_index, batch, W1, b1, g1, be1, m1, v1, W2, b2, go, bo, mo, vo, Wp, bp):
    raise NotImplementedError("write your pallas kernel here")


# speedup vs baseline: 2.5497x; 2.5497x over previous
import functools

import jax
import jax.numpy as jnp
from jax import lax
from jax.experimental import pallas as pl
from jax.experimental.pallas import tpu as pltpu
from jax.experimental.pallas import tpu_sc as plsc

N = 10000
E = 320000
D = 128
G = 256
NL = 3

NTILES = 32
ROWS_PER_TILE = 640
NPAD = 16 * ROWS_PER_TILE
TRASH = N

CW = 128
EDGES_PER_TILE = 10240
NCHUNK = EDGES_PER_TILE // CW
EPAD = NTILES * EDGES_PER_TILE

RBLK = 1024
NBLK = NPAD // RBLK


@functools.cache
def _build_sc_agg():
    mesh = plsc.VectorSubcoreMesh(core_axis_name="c", subcore_axis_name="s")

    @functools.partial(
        pl.kernel,
        out_type=jax.ShapeDtypeStruct((2, NPAD, D), jnp.float32),
        mesh=mesh,
        scratch_types=[
            pltpu.VMEM((NCHUNK, CW), jnp.int32),
            pltpu.VMEM((NCHUNK, CW), jnp.int32),
            pltpu.VMEM((CW, D), jnp.float32),
            pltpu.VMEM_SHARED((NPAD, D), jnp.float32),
            pltpu.SemaphoreType.DMA,
        ],
    )
    def sc_agg(h_hbm, src_hbm, dst_hbm, out_hbm, src_v, dst_v, buf, table,
               sem):
        c = lax.axis_index("c")
        s = lax.axis_index("s")
        wid = s * 2 + c
        row0 = s * ROWS_PER_TILE
        pltpu.sync_copy(h_hbm.at[pl.ds(row0, ROWS_PER_TILE)],
                        table.at[pl.ds(row0, ROWS_PER_TILE)])
        pltpu.sync_copy(src_hbm.at[wid], src_v)
        pltpu.sync_copy(dst_hbm.at[wid], dst_v)
        plsc.subcore_barrier()

        def body(j, carry):
            pltpu.async_copy(h_hbm.at[src_v.at[j]], buf, sem).wait()
            pltpu.sync_copy(buf, table.at[dst_v.at[j]], add=True)
            return carry

        lax.fori_loop(0, NCHUNK, body, 0)
        plsc.subcore_barrier()
        pltpu.sync_copy(table.at[pl.ds(row0, ROWS_PER_TILE)],
                        out_hbm.at[c, pl.ds(row0, ROWS_PER_TILE)])

    return sc_agg


def _mlp_block(z, W1_ref, b1_ref, g1_ref, be1_ref, m1_ref, v1_ref,
               W2_ref, b2_ref, go_ref, bo_ref, mo_ref, vo_ref):
    t = jnp.dot(z, W1_ref[...], preferred_element_type=jnp.float32) + b1_ref[...]
    s1 = g1_ref[...] * lax.rsqrt(v1_ref[...] + 1e-5)
    t = (t - m1_ref[...]) * s1 + be1_ref[...]
    t = jnp.maximum(t, 0.0)
    t = jnp.dot(t, W2_ref[...], preferred_element_type=jnp.float32) + b2_ref[...]
    so = go_ref[...] * lax.rsqrt(vo_ref[...] + 1e-5)
    t = (t - mo_ref[...]) * so + bo_ref[...]
    return jnp.maximum(t, 0.0)


def _mlp_kernel(h_ref, p_ref, W1_ref, b1_ref, g1_ref, be1_ref, m1_ref, v1_ref,
                W2_ref, b2_ref, go_ref, bo_ref, mo_ref, vo_ref, o_ref):
    z = p_ref[0] + p_ref[1] - h_ref[...]
    o_ref[...] = _mlp_block(z, W1_ref, b1_ref, g1_ref, be1_ref, m1_ref, v1_ref,
                            W2_ref, b2_ref, go_ref, bo_ref, mo_ref, vo_ref)


def _row_spec():
    return pl.BlockSpec((RBLK, D), lambda i: (i, 0))


def _full_spec(shape):
    nd = len(shape)
    return pl.BlockSpec(shape, lambda i: (0,) * nd)


_PARAM_SPECS = [_full_spec((D, D)), _full_spec((1, D)), _full_spec((1, D)),
                _full_spec((1, D)), _full_spec((1, D)), _full_spec((1, D)),
                _full_spec((D, D)), _full_spec((1, D)), _full_spec((1, D)),
                _full_spec((1, D)), _full_spec((1, D)), _full_spec((1, D))]


def _mlp_call(hp, p, params):
    return pl.pallas_call(
        _mlp_kernel,
        grid=(NBLK,),
        in_specs=[_row_spec(),
                  pl.BlockSpec((2, RBLK, D), lambda i: (0, i, 0))] + _PARAM_SPECS,
        out_specs=_row_spec(),
        out_shape=jax.ShapeDtypeStruct((NPAD, D), jnp.float32),
    )(hp, p, *params)


def _final_kernel(h_ref, p_ref, b3_ref, W1_ref, b1_ref, g1_ref, be1_ref,
                  m1_ref, v1_ref, W2_ref, b2_ref, go_ref, bo_ref, mo_ref,
                  vo_ref, Wp_ref, bp_ref, o_ref, seg_ref, cnt_ref):
    i = pl.program_id(0)
    z = p_ref[0] + p_ref[1] - h_ref[...]
    hL = _mlp_block(z, W1_ref, b1_ref, g1_ref, be1_ref, m1_ref, v1_ref,
                    W2_ref, b2_ref, go_ref, bo_ref, mo_ref, vo_ref)
    bvec = b3_ref[0, 0, :]
    gid = lax.broadcasted_iota(jnp.int32, (G, RBLK), 0)
    onehot = (gid == bvec[None, :]).astype(jnp.float32)

    @pl.when(i == 0)
    def _():
        seg_ref[...] = jnp.zeros_like(seg_ref)
        cnt_ref[...] = jnp.zeros_like(cnt_ref)

    seg_ref[...] += jnp.dot(onehot, hL, preferred_element_type=jnp.float32)
    cnt_ref[...] += jnp.sum(onehot, axis=1, keepdims=True)

    @pl.when(i == NBLK - 1)
    def _():
        mean = seg_ref[...] / jnp.maximum(cnt_ref[...], 1.0)
        o_ref[...] = (jnp.dot(mean, Wp_ref[...],
                              preferred_element_type=jnp.float32) + bp_ref[...])


def _final_call(hp, p, batch3, params, Wp, bp):
    return pl.pallas_call(
        _final_kernel,
        grid=(NBLK,),
        in_specs=[_row_spec(),
                  pl.BlockSpec((2, RBLK, D), lambda i: (0, i, 0)),
                  pl.BlockSpec((1, 1, RBLK), lambda i: (i, 0, 0))]
                 + _PARAM_SPECS + [_full_spec((D, D)), _full_spec((1, D))],
        out_specs=_full_spec((G, D)),
        out_shape=jax.ShapeDtypeStruct((G, D), jnp.float32),
        scratch_shapes=[pltpu.VMEM((G, D), jnp.float32),
                        pltpu.VMEM((G, 1), jnp.float32)],
    )(hp, p, batch3, *params, Wp, bp)


def kernel(x, edge_index, batch, W1, b1, g1, be1, m1, v1, W2, b2, go, bo, mo,
           vo, Wp, bp):
    h = x.astype(jnp.float32)
    src = edge_index[0].astype(jnp.int32)
    dst = edge_index[1].astype(jnp.int32)
    pad_e = EPAD - E
    src_p = jnp.concatenate([src, jnp.zeros((pad_e,), jnp.int32)]
                            ).reshape(NTILES, NCHUNK, CW)
    dst_p = jnp.concatenate([dst, jnp.full((pad_e,), TRASH, jnp.int32)]
                            ).reshape(NTILES, NCHUNK, CW)
    batch3 = jnp.concatenate([batch.astype(jnp.int32),
                              jnp.full((NPAD - N,), G, jnp.int32)]
                             ).reshape(NBLK, 1, RBLK)
    hp = jnp.concatenate([h, jnp.zeros((NPAD - N, D), jnp.float32)], axis=0)
    for i in range(NL):
        params = [W1[i], b1[i].reshape(1, D), g1[i].reshape(1, D),
                  be1[i].reshape(1, D), m1[i].reshape(1, D),
                  v1[i].reshape(1, D), W2[i], b2[i].reshape(1, D),
                  go[i].reshape(1, D), bo[i].reshape(1, D),
                  mo[i].reshape(1, D), vo[i].reshape(1, D)]
        p = _build_sc_agg()(hp, src_p, dst_p)
        if i < NL - 1:
            hp = _mlp_call(hp, p, params)
        else:
            out = _final_call(hp, p, batch3, params, Wp, bp.reshape(1, D))
    return out
